# Initial kernel scaffold; baseline (speedup 1.0000x reference)
import jax
import jax.numpy as jnp
from jax.experimental import pallas as pl


def kernel(loc_preds, cls_preds, targets, iou_boxes):
    raise NotImplementedError("write your pallas kernel here")


# speedup vs baseline: 3.6852x; 3.6852x over previous
import functools

import jax
import jax.numpy as jnp
from jax import lax
from jax.experimental import pallas as pl
from jax.experimental.pallas import tpu as pltpu
from jax.experimental.pallas import tpu_sc as plsc

IMG_SIZE = 600.0
ALPHA = 0.25
GAMMA = 2.0

_NC = 2
_NS = 16
_LANES = 16
_NW = _NC * _NS

_BA = 4096


def _match_body(T, CH, steps, boxs_hbm, boxg_hbm, anch_hbm, out_hbm,
                boxsv, boxgv, anchv, outv):
    wid = lax.axis_index("c") * _NS + lax.axis_index("s")
    base = wid * CH
    pltpu.sync_copy(boxs_hbm, boxsv)
    pltpu.sync_copy(boxg_hbm, boxgv)
    pltpu.sync_copy(anch_hbm.at[:, pl.ds(base, CH)], anchv)

    def step(i, carry):
        off = i * _LANES
        sl = pl.ds(off, _LANES)
        ax = anchv[0, sl]
        ay = anchv[1, sl]
        aw = anchv[2, sl]
        ah = anchv[3, sl]
        aw2 = aw * 0.5
        ah2 = ah * 0.5
        ax1 = ax - aw2
        ay1 = ay - ah2
        ax2 = ax + aw2
        ay2 = ay + ah2
        ax2p = ax2 + 1.0
        ay2p = ay2 + 1.0
        a1 = ((ax2 - ax1) + 1.0) * ((ay2 - ay1) + 1.0)

        bnum = jnp.zeros((_LANES,), jnp.float32)
        bden = jnp.ones((_LANES,), jnp.float32)
        bidx = jnp.zeros((_LANES,), jnp.int32)
        for t in range(T):
            x1t = boxsv[0, t]
            y1t = boxsv[1, t]
            x2pt = boxsv[2, t]
            y2pt = boxsv[3, t]
            at = boxsv[4, t]
            ltx = jnp.maximum(ax1, x1t)
            lty = jnp.maximum(ay1, y1t)
            rxp = jnp.minimum(ax2p, x2pt)
            ryp = jnp.minimum(ay2p, y2pt)
            wx = jnp.maximum(rxp - ltx, 0.0)
            wy = jnp.maximum(ryp - lty, 0.0)
            inter = wx * wy
            den = (a1 + at) - inter
            better = inter * bden > bnum * den
            bnum = jnp.where(better, inter, bnum)
            bden = jnp.where(better, den, bden)
            bidx = jnp.where(better, t, bidx)

        miou = bnum / bden
        lab = plsc.load_gather(boxgv, [bidx])
        bcx = plsc.load_gather(boxgv, [bidx + T])
        bcy = plsc.load_gather(boxgv, [bidx + 2 * T])
        bw = plsc.load_gather(boxgv, [bidx + 3 * T])
        bh = plsc.load_gather(boxgv, [bidx + 4 * T])

        ct = lab + 1.0
        ct = jnp.where(miou < 0.5, 0.0, ct)
        ct = jnp.where((miou > 0.4) & (miou < 0.5), -1.0, ct)
        outv[0, sl] = ct
        outv[1, sl] = (bcx - ax) / aw
        outv[2, sl] = (bcy - ay) / ah
        outv[3, sl] = bw / aw
        outv[4, sl] = bh / ah
        return carry

    lax.fori_loop(0, steps, step, 0)
    pltpu.sync_copy(outv, out_hbm.at[:, pl.ds(base, CH)])


def _loss_body(C, cls_ref, loc_ref, match_ref, out_ref):
    @pl.when(pl.program_id(0) == 0)
    def _init():
        out_ref[0] = 0.0
        out_ref[1] = 0.0
        out_ref[2] = 0.0

    ct = match_ref[0:1, :]
    pos = ct > 0.0
    posf = pos.astype(jnp.float32)
    npos = jnp.sum(posf)

    lt_enc = jnp.concatenate(
        [match_ref[1:3, :], jnp.log(match_ref[3:5, :])], axis=0)
    d = loc_ref[...] - jnp.where(pos, lt_enc, 0.0)
    ad = jnp.abs(d)
    sl1 = jnp.where(ad < 1.0, 0.5 * d * d, ad - 0.5)
    loc_sum = jnp.sum(sl1 * posf)

    x = cls_ref[...]
    cls_id = lax.broadcasted_iota(jnp.int32, x.shape, 0).astype(jnp.float32) + 1.0
    y = (cls_id == ct).astype(jnp.float32)
    p = jax.nn.sigmoid(x)
    pt = p * y + (1.0 - p) * (1.0 - y)
    w = ALPHA * y + (1.0 - ALPHA) * (1.0 - y)
    bce = jnp.maximum(x, 0.0) - x * y + jnp.log1p(jnp.exp(-jnp.abs(x)))
    focal = w * (1.0 - pt) ** GAMMA * bce
    pn = (ct > -0.5).astype(jnp.float32)
    cls_sum = jnp.sum(focal * pn)

    out_ref[0] += loc_sum
    out_ref[1] += cls_sum
    out_ref[2] += npos


def kernel(loc_preds, cls_preds, targets, iou_boxes):
    A = iou_boxes.shape[0]
    T = targets.shape[0]
    C = cls_preds.shape[-1]
    grid_n = -(-A // _BA)
    a_pad = -(-grid_n * _BA // (_NW * _LANES)) * _NW * _LANES
    grid_n = a_pad // _BA
    CH = a_pad // _NW
    steps = CH // _LANES

    boxes = targets[:, 2:] * IMG_SIZE
    labels = targets[:, 1]
    half = boxes[:, 2:] * 0.5
    xy1 = boxes[:, :2] - half
    xy2 = boxes[:, :2] + half
    area = ((xy2[:, 0] - xy1[:, 0]) + 1.0) * ((xy2[:, 1] - xy1[:, 1]) + 1.0)
    box_scan = jnp.stack(
        [xy1[:, 0], xy1[:, 1], xy2[:, 0] + 1.0, xy2[:, 1] + 1.0, area], axis=0)
    box_scan = jnp.broadcast_to(box_scan[:, :, None], (5, T, _LANES))
    box_gath = jnp.concatenate(
        [labels, boxes[:, 0], boxes[:, 1], boxes[:, 2], boxes[:, 3]], axis=0)

    npad = a_pad - A
    anch_pad = jnp.tile(
        jnp.array([[-1e6], [-1e6], [1.0], [1.0]], jnp.float32), (1, npad))
    anchT = jnp.concatenate([iou_boxes.T, anch_pad], axis=1)
    clsT = jnp.concatenate(
        [cls_preds[0].T, jnp.full((C, npad), -1e30, jnp.float32)], axis=1)
    locT = jnp.concatenate(
        [loc_preds[0].T, jnp.zeros((4, npad), jnp.float32)], axis=1)

    mesh = plsc.VectorSubcoreMesh(
        core_axis_name="c", subcore_axis_name="s",
        num_cores=_NC, num_subcores=_NS)
    match = pl.kernel(
        functools.partial(_match_body, T, CH, steps),
        out_type=jax.ShapeDtypeStruct((5, a_pad), jnp.float32),
        mesh=mesh,
        scratch_types=[
            pltpu.VMEM((5, T, _LANES), jnp.float32),
            pltpu.VMEM((5 * T,), jnp.float32),
            pltpu.VMEM((4, CH), jnp.float32),
            pltpu.VMEM((5, CH), jnp.float32),
        ],
        compiler_params=pltpu.CompilerParams(needs_layout_passes=False),
    )(box_scan, box_gath, anchT)

    sums = pl.pallas_call(
        functools.partial(_loss_body, C),
        grid=(grid_n,),
        in_specs=[
            pl.BlockSpec((C, _BA), lambda i: (0, i)),
            pl.BlockSpec((4, _BA), lambda i: (0, i)),
            pl.BlockSpec((5, _BA), lambda i: (0, i)),
        ],
        out_specs=pl.BlockSpec(memory_space=pltpu.SMEM),
        out_shape=jax.ShapeDtypeStruct((3,), jnp.float32),
    )(clsT, locT, match)

    num_pos = jnp.maximum(1.0, sums[2])
    loc_part = sums[0] / num_pos
    cls_part = sums[1] / num_pos
    return (loc_part + cls_part, loc_part, cls_part)


# speedup vs baseline: 3.6992x; 1.0038x over previous
import functools

import jax
import jax.numpy as jnp
from jax import lax
from jax.experimental import pallas as pl
from jax.experimental.pallas import tpu as pltpu
from jax.experimental.pallas import tpu_sc as plsc

IMG_SIZE = 600.0
ALPHA = 0.25
GAMMA = 2.0

_NC = 2
_NS = 16
_LANES = 16
_NW = _NC * _NS

_BA = 4096


def _match_body(T, CH, steps, boxs_hbm, boxg_hbm, anch_hbm, out_hbm,
                boxsv, boxgv, anchv, outv):
    wid = lax.axis_index("c") * _NS + lax.axis_index("s")
    base = wid * CH
    pltpu.sync_copy(boxs_hbm, boxsv)
    pltpu.sync_copy(boxg_hbm, boxgv)
    pltpu.sync_copy(anch_hbm.at[:, pl.ds(base, CH)], anchv)

    def step(i, carry):
        off = i * _LANES
        sl = pl.ds(off, _LANES)
        ax = anchv[0, sl]
        ay = anchv[1, sl]
        aw = anchv[2, sl]
        ah = anchv[3, sl]
        aw2 = aw * 0.5
        ah2 = ah * 0.5
        ax1 = ax - aw2
        ay1 = ay - ah2
        ax2 = ax + aw2
        ay2 = ay + ah2
        ax2p = ax2 + 1.0
        ay2p = ay2 + 1.0
        a1 = ((ax2 - ax1) + 1.0) * ((ay2 - ay1) + 1.0)

        bnum = jnp.zeros((_LANES,), jnp.float32)
        bden = jnp.ones((_LANES,), jnp.float32)
        bidx = jnp.zeros((_LANES,), jnp.int32)
        for t in range(T):
            x1t = boxsv[0, t]
            y1t = boxsv[1, t]
            x2pt = boxsv[2, t]
            y2pt = boxsv[3, t]
            at = boxsv[4, t]
            ltx = jnp.maximum(ax1, x1t)
            lty = jnp.maximum(ay1, y1t)
            rxp = jnp.minimum(ax2p, x2pt)
            ryp = jnp.minimum(ay2p, y2pt)
            wx = jnp.maximum(rxp - ltx, 0.0)
            wy = jnp.maximum(ryp - lty, 0.0)
            inter = wx * wy
            den = (a1 + at) - inter
            better = inter * bden > bnum * den
            bnum = jnp.where(better, inter, bnum)
            bden = jnp.where(better, den, bden)
            bidx = jnp.where(better, t, bidx)

        miou = bnum / bden
        lab = plsc.load_gather(boxgv, [bidx])
        bcx = plsc.load_gather(boxgv, [bidx + T])
        bcy = plsc.load_gather(boxgv, [bidx + 2 * T])
        bw = plsc.load_gather(boxgv, [bidx + 3 * T])
        bh = plsc.load_gather(boxgv, [bidx + 4 * T])

        ct = lab + 1.0
        ct = jnp.where(miou < 0.5, 0.0, ct)
        ct = jnp.where((miou > 0.4) & (miou < 0.5), -1.0, ct)
        outv[0, sl] = ct
        outv[1, sl] = (bcx - ax) / aw
        outv[2, sl] = (bcy - ay) / ah
        outv[3, sl] = bw / aw
        outv[4, sl] = bh / ah
        return carry

    lax.fori_loop(0, steps, step, 0)
    pltpu.sync_copy(outv, out_hbm.at[:, pl.ds(base, CH)])


def _fold_lanes(v, width):
    while width > 128:
        width //= 2
        v = v[:, :width] + v[:, width:]
    return v


def _f01(x):
    e = jnp.exp(-jnp.abs(x))
    l1p = jnp.log1p(e)
    s = jnp.maximum(x, 0.0) + l1p
    r = 1.0 / (1.0 + e)
    p = jnp.where(x >= 0.0, r, e * r)
    f0 = (1.0 - ALPHA) * (p * p) * s
    q = 1.0 - p
    f1 = ALPHA * (q * q) * (s - x)
    return f0, f1


def _loss_body(C, nsteps, cls_ref, loc_ref, match_ref, out_ref, acc):
    @pl.when(pl.program_id(0) == 0)
    def _init():
        acc[...] = jnp.zeros_like(acc)

    ct = match_ref[0:8, :]
    pos = ct > 0.0
    posf = pos.astype(jnp.float32)
    pn = (ct > -0.5).astype(jnp.float32)

    sl1_sum = None
    for j in range(4):
        enc = match_ref[8 * (j + 1):8 * (j + 2), :]
        if j >= 2:
            enc = jnp.log(enc)
        d = loc_ref[8 * j:8 * (j + 1), :] - jnp.where(pos, enc, 0.0)
        ad = jnp.abs(d)
        sl1 = jnp.where(ad < 1.0, 0.5 * d * d, ad - 0.5)
        sl1_sum = sl1 if sl1_sum is None else sl1_sum + sl1
    loc_v = sl1_sum * posf

    x = cls_ref[...]
    f0, _ = _f01(x)
    s0 = None
    xsel = None
    for c in range(C):
        blk = slice(8 * c, 8 * (c + 1))
        s0 = f0[blk, :] if s0 is None else s0 + f0[blk, :]
        xc = jnp.where(ct == float(c + 1), x[blk, :], 0.0)
        xsel = xc if xsel is None else xsel + xc
    g0, g1 = _f01(xsel)
    cls_v = pn * s0 + posf * (g1 - g0)

    acc[0:8, :] += _fold_lanes(cls_v, cls_v.shape[1])
    acc[8:16, :] += _fold_lanes(loc_v, loc_v.shape[1])
    acc[16:24, :] += _fold_lanes(posf, posf.shape[1])

    @pl.when(pl.program_id(0) == nsteps - 1)
    def _fin():
        out_ref[0] = jnp.sum(acc[8:16, :])
        out_ref[1] = jnp.sum(acc[0:8, :])
        out_ref[2] = jnp.sum(acc[16:24, :])


def kernel(loc_preds, cls_preds, targets, iou_boxes):
    A = iou_boxes.shape[0]
    T = targets.shape[0]
    C = cls_preds.shape[-1]
    grid_n = -(-A // _BA)
    a_pad = -(-grid_n * _BA // (_NW * _LANES)) * _NW * _LANES
    grid_n = a_pad // _BA
    CH = a_pad // _NW
    steps = CH // _LANES

    boxes = targets[:, 2:] * IMG_SIZE
    labels = targets[:, 1]
    half = boxes[:, 2:] * 0.5
    xy1 = boxes[:, :2] - half
    xy2 = boxes[:, :2] + half
    area = ((xy2[:, 0] - xy1[:, 0]) + 1.0) * ((xy2[:, 1] - xy1[:, 1]) + 1.0)
    box_scan = jnp.stack(
        [xy1[:, 0], xy1[:, 1], xy2[:, 0] + 1.0, xy2[:, 1] + 1.0, area], axis=0)
    box_scan = jnp.broadcast_to(box_scan[:, :, None], (5, T, _LANES))
    box_gath = jnp.concatenate(
        [labels, boxes[:, 0], boxes[:, 1], boxes[:, 2], boxes[:, 3]], axis=0)

    npad = a_pad - A
    anch_pad = jnp.tile(
        jnp.array([[-1e6], [-1e6], [1.0], [1.0]], jnp.float32), (1, npad))
    anchT = jnp.concatenate([iou_boxes.T, anch_pad], axis=1)
    clsT = jnp.concatenate(
        [cls_preds[0].T, jnp.full((C, npad), -1e30, jnp.float32)], axis=1)
    locT = jnp.concatenate(
        [loc_preds[0].T, jnp.zeros((4, npad), jnp.float32)], axis=1)

    mesh = plsc.VectorSubcoreMesh(
        core_axis_name="c", subcore_axis_name="s",
        num_cores=_NC, num_subcores=_NS)
    match = pl.kernel(
        functools.partial(_match_body, T, CH, steps),
        out_type=jax.ShapeDtypeStruct((5, a_pad), jnp.float32),
        mesh=mesh,
        scratch_types=[
            pltpu.VMEM((5, T, _LANES), jnp.float32),
            pltpu.VMEM((5 * T,), jnp.float32),
            pltpu.VMEM((4, CH), jnp.float32),
            pltpu.VMEM((5, CH), jnp.float32),
        ],
        compiler_params=pltpu.CompilerParams(needs_layout_passes=False),
    )(box_scan, box_gath, anchT)

    a8 = a_pad // 8
    bl = _BA // 8
    sums = pl.pallas_call(
        functools.partial(_loss_body, C, grid_n),
        grid=(grid_n,),
        in_specs=[
            pl.BlockSpec((C * 8, bl), lambda i: (0, i)),
            pl.BlockSpec((4 * 8, bl), lambda i: (0, i)),
            pl.BlockSpec((5 * 8, bl), lambda i: (0, i)),
        ],
        out_specs=pl.BlockSpec(memory_space=pltpu.SMEM),
        out_shape=jax.ShapeDtypeStruct((3,), jnp.float32),
        scratch_shapes=[pltpu.VMEM((24, 128), jnp.float32)],
    )(clsT.reshape(C * 8, a8), locT.reshape(4 * 8, a8),
      match.reshape(5 * 8, a8))

    num_pos = jnp.maximum(1.0, sums[2])
    loc_part = sums[0] / num_pos
    cls_part = sums[1] / num_pos
    return (loc_part + cls_part, loc_part, cls_part)


# speedup vs baseline: 4.0467x; 1.0939x over previous
import functools

import jax
import jax.numpy as jnp
from jax import lax
from jax.experimental import pallas as pl
from jax.experimental.pallas import tpu as pltpu
from jax.experimental.pallas import tpu_sc as plsc

IMG_SIZE = 600.0
ALPHA = 0.25
GAMMA = 2.0

_NC = 2
_NS = 16
_LANES = 16
_NW = _NC * _NS

_BA = 4096


def _match_body(T, CH, steps, boxs_hbm, boxg_hbm, anch_hbm, out_hbm,
                boxsv, boxgv, anchv, outv):
    wid = lax.axis_index("c") * _NS + lax.axis_index("s")
    base = wid * CH
    pltpu.sync_copy(boxs_hbm, boxsv)
    pltpu.sync_copy(boxg_hbm, boxgv)
    pltpu.sync_copy(anch_hbm.at[:, pl.ds(base, CH)], anchv)

    def step(i, carry):
        off = i * _LANES
        sl = pl.ds(off, _LANES)
        ax = anchv[0, sl]
        ay = anchv[1, sl]
        aw = anchv[2, sl]
        ah = anchv[3, sl]
        aw2 = aw * 0.5
        ah2 = ah * 0.5
        ax1 = ax - aw2
        ay1 = ay - ah2
        ax2 = ax + aw2
        ay2 = ay + ah2
        ax2p = ax2 + 1.0
        ay2p = ay2 + 1.0
        a1 = ((ax2 - ax1) + 1.0) * ((ay2 - ay1) + 1.0)

        bnum = jnp.zeros((_LANES,), jnp.float32)
        bden = jnp.ones((_LANES,), jnp.float32)
        bidx = jnp.zeros((_LANES,), jnp.int32)
        for t in range(T):
            x1t = boxsv[0, t]
            y1t = boxsv[1, t]
            x2pt = boxsv[2, t]
            y2pt = boxsv[3, t]
            at = boxsv[4, t]
            ltx = jnp.maximum(ax1, x1t)
            lty = jnp.maximum(ay1, y1t)
            rxp = jnp.minimum(ax2p, x2pt)
            ryp = jnp.minimum(ay2p, y2pt)
            wx = jnp.maximum(rxp - ltx, 0.0)
            wy = jnp.maximum(ryp - lty, 0.0)
            inter = wx * wy
            den = (a1 + at) - inter
            better = inter * bden > bnum * den
            bnum = jnp.where(better, inter, bnum)
            bden = jnp.where(better, den, bden)
            bidx = jnp.where(better, t, bidx)

        miou = bnum / bden
        lab = plsc.load_gather(boxgv, [bidx])
        bcx = plsc.load_gather(boxgv, [bidx + T])
        bcy = plsc.load_gather(boxgv, [bidx + 2 * T])
        bw = plsc.load_gather(boxgv, [bidx + 3 * T])
        bh = plsc.load_gather(boxgv, [bidx + 4 * T])

        ct = lab + 1.0
        ct = jnp.where(miou < 0.5, 0.0, ct)
        ct = jnp.where((miou > 0.4) & (miou < 0.5), -1.0, ct)
        outv[0, sl] = ct
        outv[1, sl] = (bcx - ax) / aw
        outv[2, sl] = (bcy - ay) / ah
        outv[3, sl] = bw / aw
        outv[4, sl] = bh / ah
        return carry

    lax.fori_loop(0, steps, step, 0)
    pltpu.sync_copy(outv, out_hbm.at[:, pl.ds(base, CH)])


def _fold_lanes(v, width):
    while width > 128:
        width //= 2
        v = v[:, :width] + v[:, width:]
    return v


def _f01(x):
    e = jnp.exp(-jnp.abs(x))
    l1p = jnp.log1p(e)
    s = jnp.maximum(x, 0.0) + l1p
    r = 1.0 / (1.0 + e)
    p = jnp.where(x >= 0.0, r, e * r)
    f0 = (1.0 - ALPHA) * (p * p) * s
    q = 1.0 - p
    f1 = ALPHA * (q * q) * (s - x)
    return f0, f1


def _loss_body(C, A, BA, nsteps, cls_ref, loc_ref, match_ref, out_ref, acc):
    @pl.when(pl.program_id(0) == 0)
    def _init():
        acc[...] = jnp.zeros_like(acc)

    ct = match_ref[0:1, :]
    pos = ct > 0.0
    posf = pos.astype(jnp.float32)
    pn = (ct > -0.5).astype(jnp.float32)
    lane = lax.broadcasted_iota(jnp.int32, (1, BA), 1)
    valid = (pl.program_id(0) * BA + lane) < A

    enc_xy = match_ref[1:3, :]
    enc_wh = jnp.log(match_ref[3:5, :])
    enc = jnp.concatenate([enc_xy, enc_wh], axis=0)
    d = loc_ref[...] - jnp.where(pos, enc, 0.0)
    ad = jnp.abs(d)
    sl1 = jnp.where(ad < 1.0, 0.5 * d * d, ad - 0.5)
    loc_v = jnp.where(pos, sl1, 0.0)

    x = cls_ref[...]
    f0, f1 = _f01(x)
    cls_id = lax.broadcasted_iota(
        jnp.int32, x.shape, 0).astype(jnp.float32) + 1.0
    f = f0 + jnp.where(cls_id == ct, f1 - f0, 0.0)
    f = jnp.where(valid, f, 0.0)
    t8 = f[0:8, :] + f[8:16, :]
    v4 = (t8[0:4, :] + t8[4:8, :]) + f[16:C, :]
    cls_v = v4 * pn

    acc[0:4, :] += _fold_lanes(cls_v, cls_v.shape[1])
    acc[4:8, :] += _fold_lanes(loc_v, loc_v.shape[1])
    acc[8:9, :] += _fold_lanes(posf, posf.shape[1])

    @pl.when(pl.program_id(0) == nsteps - 1)
    def _fin():
        loc_s = jnp.sum(acc[4:8, :])
        cls_s = jnp.sum(acc[0:4, :])
        np_s = jnp.maximum(1.0, jnp.sum(acc[8:9, :]))
        out_ref[0] = (loc_s + cls_s) / np_s
        out_ref[1] = loc_s / np_s
        out_ref[2] = cls_s / np_s


def kernel(loc_preds, cls_preds, targets, iou_boxes):
    A = iou_boxes.shape[0]
    T = targets.shape[0]
    C = cls_preds.shape[-1]
    grid_n = -(-A // _BA)
    a_pad = -(-grid_n * _BA // (_NW * _LANES)) * _NW * _LANES
    grid_n = a_pad // _BA
    CH = a_pad // _NW
    steps = CH // _LANES

    boxes = targets[:, 2:] * IMG_SIZE
    labels = targets[:, 1]
    half = boxes[:, 2:] * 0.5
    xy1 = boxes[:, :2] - half
    xy2 = boxes[:, :2] + half
    area = ((xy2[:, 0] - xy1[:, 0]) + 1.0) * ((xy2[:, 1] - xy1[:, 1]) + 1.0)
    box_scan = jnp.stack(
        [xy1[:, 0], xy1[:, 1], xy2[:, 0] + 1.0, xy2[:, 1] + 1.0, area], axis=0)
    box_scan = jnp.broadcast_to(box_scan[:, :, None], (5, T, _LANES))
    box_gath = jnp.concatenate(
        [labels, boxes[:, 0], boxes[:, 1], boxes[:, 2], boxes[:, 3]], axis=0)

    npad = a_pad - A
    anch_pad = jnp.tile(
        jnp.array([[-1e6], [-1e6], [1.0], [1.0]], jnp.float32), (1, npad))
    anchT = jnp.concatenate([iou_boxes.T, anch_pad], axis=1)
    clsT = cls_preds[0].T
    locT = loc_preds[0].T

    mesh = plsc.VectorSubcoreMesh(
        core_axis_name="c", subcore_axis_name="s",
        num_cores=_NC, num_subcores=_NS)
    match = pl.kernel(
        functools.partial(_match_body, T, CH, steps),
        out_type=jax.ShapeDtypeStruct((5, a_pad), jnp.float32),
        mesh=mesh,
        scratch_types=[
            pltpu.VMEM((5, T, _LANES), jnp.float32),
            pltpu.VMEM((5 * T,), jnp.float32),
            pltpu.VMEM((4, CH), jnp.float32),
            pltpu.VMEM((5, CH), jnp.float32),
        ],
        compiler_params=pltpu.CompilerParams(needs_layout_passes=False),
    )(box_scan, box_gath, anchT)

    sums = pl.pallas_call(
        functools.partial(_loss_body, C, A, _BA, grid_n),
        grid=(grid_n,),
        in_specs=[
            pl.BlockSpec((C, _BA), lambda i: (0, i)),
            pl.BlockSpec((4, _BA), lambda i: (0, i)),
            pl.BlockSpec((5, _BA), lambda i: (0, i)),
        ],
        out_specs=pl.BlockSpec(memory_space=pltpu.SMEM),
        out_shape=jax.ShapeDtypeStruct((3,), jnp.float32),
        scratch_shapes=[pltpu.VMEM((16, 128), jnp.float32)],
    )(clsT, locT, match)

    return (sums[0], sums[1], sums[2])
